# Initial kernel scaffold; baseline (speedup 1.0000x reference)
import jax
import jax.numpy as jnp
from jax.experimental import pallas as pl


def kernel(input_data, markers, hidden, W, b, W_out, b_out):
    raise NotImplementedError("write your pallas kernel here")


# speedup vs baseline: 3.3550x; 3.3550x over previous
import functools

import jax
import jax.numpy as jnp
from jax.experimental import pallas as pl
from jax.experimental.pallas import tpu as pltpu


def _rnn_kernel(markers_ref, data_ref, W_ref, b_ref, hid_ref, Wout_ref,
                bout_ref, out_ref, hout_ref, g_ref, i_ref):
    l = pl.program_id(0)
    nlayers = pl.num_programs(0)

    @pl.when(l == 0)
    def _init():
        i_ref[0] = 0
        g_ref[1:1025, :] = hid_ref[...]

    cnt = markers_ref[0, l]

    def body(_, carry):
        i = i_ref[0]
        g_ref[0:1, 0:1] = jnp.full((1, 1), data_ref[0, i], jnp.float32)
        t = jax.lax.dot_general(
            W_ref[0], g_ref[...],
            (((1,), (0,)), ((), ())),
            preferred_element_type=jnp.float32)
        h_new = jnp.maximum(t + b_ref[0], 0.0)
        g_ref[1:1025, :] = h_new
        i_ref[0] = i + 1
        return carry

    jax.lax.fori_loop(0, cnt, body, 0, unroll=False)

    @pl.when(l == nlayers - 1)
    def _finish():
        h_fin = g_ref[1:1025, :]
        logits = jax.lax.dot_general(
            Wout_ref[...], h_fin,
            (((1,), (0,)), ((), ())),
            preferred_element_type=jnp.float32) + bout_ref[...]
        m = jnp.max(logits)
        z = logits - m
        out_ref[...] = z - jnp.log(jnp.sum(jnp.exp(z)))
        hout_ref[...] = h_fin


@jax.jit
def kernel(input_data, markers, hidden, W, b, W_out, b_out):
    nlayers, hid, inpp1 = W.shape
    out_sz = W_out.shape[0]

    b_col = b[:, :, None]
    bout_col = b_out[:, None]
    hid_col = hidden.reshape(hid, 1)

    grid = (nlayers,)
    out_col, h_col = pl.pallas_call(
        _rnn_kernel,
        grid=grid,
        in_specs=[
            pl.BlockSpec(memory_space=pltpu.SMEM),
            pl.BlockSpec(memory_space=pltpu.SMEM),
            pl.BlockSpec((1, hid, inpp1), lambda l: (l, 0, 0)),
            pl.BlockSpec((1, hid, 1), lambda l: (l, 0, 0)),
            pl.BlockSpec((hid, 1), lambda l: (0, 0)),
            pl.BlockSpec((out_sz, hid), lambda l: (0, 0)),
            pl.BlockSpec((out_sz, 1), lambda l: (0, 0)),
        ],
        out_specs=[
            pl.BlockSpec((out_sz, 1), lambda l: (0, 0)),
            pl.BlockSpec((hid, 1), lambda l: (0, 0)),
        ],
        out_shape=[
            jax.ShapeDtypeStruct((out_sz, 1), jnp.float32),
            jax.ShapeDtypeStruct((hid, 1), jnp.float32),
        ],
        scratch_shapes=[
            pltpu.VMEM((inpp1, 1), jnp.float32),
            pltpu.SMEM((1,), jnp.int32),
        ],
    )(markers, input_data, W, b_col, hid_col, W_out, bout_col)

    return out_col.reshape(1, out_sz), h_col.reshape(1, hid)
